# baseline (speedup 1.0000x reference)
import functools

import jax
import jax.numpy as jnp
from jax import lax
from jax.experimental import pallas as pl
from jax.experimental.pallas import tpu as pltpu
from jax.experimental.pallas import tpu_sc as plsc

N, E, F, D, G = 10000, 320000, 128, 16, 64

_NC, _NS = 2, 16
_NW = _NC * _NS
_CPW = E // _NW
_KG = 400
_NCH = _CPW // _KG
_ZPS = 624
_ZTAIL = N - _ZPS * _NS
_LANES = 16
_KS = 80
_NCHS = _CPW // _KS


def _sc_mesh():
    return plsc.VectorSubcoreMesh(
        core_axis_name="c", subcore_axis_name="s",
        num_cores=_NC, num_subcores=_NS)


def _sc_gather2(h, dst, src, ne, kg):
    cpw = ne // _NW
    nch = cpw // kg

    @functools.partial(
        pl.kernel,
        out_type=[jax.ShapeDtypeStruct((ne, F), jnp.float32),
                  jax.ShapeDtypeStruct((ne, F), jnp.float32)],
        mesh=_sc_mesh(),
        scratch_types=[
            pltpu.VMEM((kg,), jnp.int32),
            pltpu.VMEM((kg,), jnp.int32),
            pltpu.VMEM((kg, F), jnp.float32),
            pltpu.VMEM((kg, F), jnp.float32),
            pltpu.VMEM_SHARED((N, F), jnp.float32),
            pltpu.SemaphoreType.DMA,
            pltpu.SemaphoreType.DMA,
            pltpu.SemaphoreType.DMA,
            pltpu.SemaphoreType.DMA,
        ],
    )
    def k(h_hbm, d_hbm, s_hbm, od_hbm, os_hbm, idxd, idxs, bufd, bufs,
          table, semgd, semgs, semod, semos):
        sid = lax.axis_index("s")
        wid = sid * _NC + lax.axis_index("c")
        base = wid * cpw

        _stage_rows(h_hbm, table, sid)
        plsc.subcore_barrier()

        pltpu.sync_copy(d_hbm.at[pl.ds(base, kg)], idxd)
        pltpu.async_copy(table.at[idxd], bufd, semgd)
        pltpu.sync_copy(s_hbm.at[pl.ds(base, kg)], idxs)
        pltpu.async_copy(table.at[idxs], bufs, semgs)

        @pl.loop(0, nch)
        def _(j):
            off = base + j * kg
            pltpu.make_async_copy(table.at[idxd], bufd, semgd).wait()
            pltpu.async_copy(bufd, od_hbm.at[pl.ds(off, kg)], semod)
            pltpu.make_async_copy(table.at[idxs], bufs, semgs).wait()
            pltpu.async_copy(bufs, os_hbm.at[pl.ds(off, kg)], semos)

            @pl.when(j < nch - 1)
            def _():
                nxt = off + kg
                pltpu.sync_copy(d_hbm.at[pl.ds(nxt, kg)], idxd)
                pltpu.make_async_copy(
                    bufd, od_hbm.at[pl.ds(off, kg)], semod).wait()
                pltpu.async_copy(table.at[idxd], bufd, semgd)
                pltpu.sync_copy(s_hbm.at[pl.ds(nxt, kg)], idxs)
                pltpu.make_async_copy(
                    bufs, os_hbm.at[pl.ds(off, kg)], semos).wait()
                pltpu.async_copy(table.at[idxs], bufs, semgs)

        last = base + (nch - 1) * kg
        pltpu.make_async_copy(bufd, od_hbm.at[pl.ds(last, kg)], semod).wait()
        pltpu.make_async_copy(bufs, os_hbm.at[pl.ds(last, kg)], semos).wait()

    return k(h, dst, src)


def _stage_rows(src_hbm, shared, sid):
    row0 = sid * _ZPS
    pltpu.sync_copy(src_hbm.at[pl.ds(row0, _ZPS)],
                    shared.at[pl.ds(row0, _ZPS)])

    @pl.when(sid == _NS - 1)
    def _():
        pltpu.sync_copy(src_hbm.at[pl.ds(_ZPS * _NS, _ZTAIL)],
                        shared.at[pl.ds(_ZPS * _NS, _ZTAIL)])


def _zero_acc(z_hbm, acc, sid):
    row0 = sid * _ZPS
    pltpu.sync_copy(z_hbm.at[pl.ds(row0, _ZPS)], acc.at[pl.ds(row0, _ZPS)])

    @pl.when(sid == _NS - 1)
    def _():
        pltpu.sync_copy(z_hbm.at[pl.ds(_ZPS * _NS, _ZTAIL)],
                        acc.at[pl.ds(_ZPS * _NS, _ZTAIL)])


def _writeback(acc, out_hbm, cid, sid):
    row0 = sid * _ZPS
    pltpu.sync_copy(acc.at[pl.ds(row0, _ZPS)],
                    out_hbm.at[cid].at[pl.ds(row0, _ZPS)])

    @pl.when(sid == _NS - 1)
    def _():
        pltpu.sync_copy(
            acc.at[pl.ds(_ZPS * _NS, _ZTAIL)],
            out_hbm.at[cid].at[pl.ds(_ZPS * _NS, _ZTAIL)])


def _sc_scatter_add(m, dst, zeros_acc, ne, ks):
    cpw = ne // _NW
    nchs = cpw // ks

    @functools.partial(
        pl.kernel,
        out_type=jax.ShapeDtypeStruct((_NC, N, F), jnp.float32),
        mesh=_sc_mesh(),
        scratch_types=[
            [pltpu.VMEM((ks,), jnp.int32)] * 2,
            [pltpu.VMEM((ks, F), jnp.float32)] * 2,
            pltpu.VMEM_SHARED((N, F), jnp.float32),
            [pltpu.SemaphoreType.DMA] * 2,
        ],
    )
    def k(m_hbm, d_hbm, z_hbm, out_hbm, idx, buf, acc, semm):
        cid = lax.axis_index("c")
        sid = lax.axis_index("s")
        base = (sid * _NC + cid) * cpw

        _zero_acc(z_hbm, acc, sid)
        plsc.subcore_barrier()

        for b in (0, 1):
            pltpu.async_copy(m_hbm.at[pl.ds(base + b * ks, ks)],
                             buf[b], semm[b])

        def step(j, b):
            off = base + j * ks
            pltpu.sync_copy(d_hbm.at[pl.ds(off, ks)], idx[b])
            pltpu.make_async_copy(
                m_hbm.at[pl.ds(off, ks)], buf[b], semm[b]).wait()
            pltpu.sync_copy(buf[b], acc.at[idx[b]], add=True)

            @pl.when(j + 2 < nchs)
            def _():
                pltpu.async_copy(
                    m_hbm.at[pl.ds(off + 2 * ks, ks)], buf[b], semm[b])

        @pl.loop(0, nchs // 2)
        def _(t):
            step(2 * t, 0)
            step(2 * t + 1, 1)

        if nchs % 2:
            step(nchs - 1, 0)

        plsc.subcore_barrier()
        _writeback(acc, out_hbm, cid, sid)

    return k(m, dst, zeros_acc)


def _sc_gather_scatter(h1, src, dst, zeros_acc):
    @functools.partial(
        pl.kernel,
        out_type=jax.ShapeDtypeStruct((_NC, N, F), jnp.float32),
        mesh=_sc_mesh(),
        scratch_types=[
            [pltpu.VMEM((_KS,), jnp.int32)] * 2,
            pltpu.VMEM((_KS,), jnp.int32),
            [pltpu.VMEM((_KS, F), jnp.float32)] * 2,
            pltpu.VMEM_SHARED((N, F), jnp.float32),
            [pltpu.SemaphoreType.DMA] * 2,
        ],
    )
    def k(h_hbm, s_hbm, d_hbm, z_hbm, out_hbm, idxs, idxd, buf, acc, semg):
        cid = lax.axis_index("c")
        sid = lax.axis_index("s")
        base = (sid * _NC + cid) * _CPW

        _zero_acc(z_hbm, acc, sid)
        plsc.subcore_barrier()

        for b in (0, 1):
            pltpu.sync_copy(s_hbm.at[pl.ds(base + b * _KS, _KS)], idxs[b])
            pltpu.async_copy(h_hbm.at[idxs[b]], buf[b], semg[b])

        def step(j, b):
            off = base + j * _KS
            pltpu.sync_copy(d_hbm.at[pl.ds(off, _KS)], idxd)
            pltpu.make_async_copy(
                h_hbm.at[idxs[b]], buf[b], semg[b]).wait()
            pltpu.sync_copy(buf[b], acc.at[idxd], add=True)

            @pl.when(j + 2 < _NCHS)
            def _():
                pltpu.sync_copy(
                    s_hbm.at[pl.ds(off + 2 * _KS, _KS)], idxs[b])
                pltpu.async_copy(h_hbm.at[idxs[b]], buf[b], semg[b])

        @pl.loop(0, _NCHS // 2)
        def _(t):
            step(2 * t, 0)
            step(2 * t + 1, 1)

        step(_NCHS - 1, 0)

        plsc.subcore_barrier()
        _writeback(acc, out_hbm, cid, sid)

    return k(h1, src, dst, zeros_acc)


_BN = 2000
_BE = 2000


def _sigmoid(x):
    return 1.0 / (1.0 + jnp.exp(-x))


def _softplus(x):
    return jnp.maximum(x, 0.0) + jnp.log(1.0 + jnp.exp(-jnp.abs(x)))


def _emb_body(x_ref, w_ref, b_ref, o_ref):
    o_ref[...] = (
        jnp.dot(x_ref[...], w_ref[...], preferred_element_type=jnp.float32)
        + b_ref[...]
    )


def _emb(x, W, b):
    return pl.pallas_call(
        _emb_body,
        grid=(N // _BN,),
        in_specs=[
            pl.BlockSpec((_BN, F), lambda i: (i, 0)),
            pl.BlockSpec((F, F), lambda i: (0, 0)),
            pl.BlockSpec((1, F), lambda i: (0, 0)),
        ],
        out_specs=pl.BlockSpec((_BN, F), lambda i: (i, 0)),
        out_shape=jax.ShapeDtypeStruct((N, F), jnp.float32),
    )(x, W, b.reshape(1, F))


def _edge_body(hd_ref, hs_ref, e_ref, wf_ref, bf_ref, ws_ref, bs_ref, m_ref):
    hd = hd_ref[...]
    hs = hs_ref[...]
    ea = e_ref[...]

    def proj(w_ref, b_ref):
        return (
            jnp.dot(hd, w_ref[0:F, :], preferred_element_type=jnp.float32)
            + jnp.dot(hs, w_ref[F:2 * F, :], preferred_element_type=jnp.float32)
            + jnp.dot(ea, w_ref[2 * F:2 * F + D, :],
                      preferred_element_type=jnp.float32)
            + b_ref[...]
        )

    f = proj(wf_ref, bf_ref)
    s = proj(ws_ref, bs_ref)
    m_ref[...] = _sigmoid(f) * _softplus(s)


def _edge_messages(hd, hs, ea, Wf, bf, Ws, bs, ne):
    return pl.pallas_call(
        _edge_body,
        grid=(ne // _BE,),
        in_specs=[
            pl.BlockSpec((_BE, F), lambda i: (i, 0)),
            pl.BlockSpec((_BE, F), lambda i: (i, 0)),
            pl.BlockSpec((_BE, D), lambda i: (i, 0)),
            pl.BlockSpec((2 * F + D, F), lambda i: (0, 0)),
            pl.BlockSpec((1, F), lambda i: (0, 0)),
            pl.BlockSpec((2 * F + D, F), lambda i: (0, 0)),
            pl.BlockSpec((1, F), lambda i: (0, 0)),
        ],
        out_specs=pl.BlockSpec((_BE, F), lambda i: (i, 0)),
        out_shape=jax.ShapeDtypeStruct((ne, F), jnp.float32),
    )(hd, hs, ea, Wf, bf.reshape(1, F), Ws, bs.reshape(1, F))


def _resid_body(h_ref, p0_ref, p1_ref, p2_ref, p3_ref, o_ref):
    o_ref[...] = jnp.maximum(
        h_ref[...] + (p0_ref[...] + p1_ref[...])
        + (p2_ref[...] + p3_ref[...]), 0.0)


def _resid_relu(h, pa, pb):
    return pl.pallas_call(
        _resid_body,
        grid=(N // _BN,),
        in_specs=[pl.BlockSpec((_BN, F), lambda i: (i, 0))] * 5,
        out_specs=pl.BlockSpec((_BN, F), lambda i: (i, 0)),
        out_shape=jax.ShapeDtypeStruct((N, F), jnp.float32),
    )(h, pa[0], pa[1], pb[0], pb[1])


def _sag_body(h1_ref, a0_ref, a1_ref, b_ref, wrel_ref, brel_ref, wroot_ref,
              h2_ref, oc_ref):
    h1 = h1_ref[...]
    agg = a0_ref[...] + a1_ref[...]
    bcol = b_ref[...]
    s = (
        jnp.dot(agg, wrel_ref[...], preferred_element_type=jnp.float32)
        + brel_ref[...]
        + jnp.dot(h1, wroot_ref[...], preferred_element_type=jnp.float32)
    )
    seg = jax.lax.broadcasted_iota(jnp.int32, (N, G), 1)
    mask = bcol == seg
    maskf = mask.astype(jnp.float32)
    neg = jnp.float32(-1e30)
    smax = jnp.max(jnp.where(mask, s, neg), axis=0, keepdims=True)
    smax_n = jnp.sum(maskf * smax, axis=1, keepdims=True)
    ex = jnp.exp(s - smax_n)
    den = jnp.sum(maskf * ex, axis=0, keepdims=True)
    den_n = jnp.sum(maskf * den, axis=1, keepdims=True)
    h2 = h1 * (ex / den_n)
    h2_ref[...] = h2
    sum_g = jax.lax.dot_general(
        maskf, h2, (((0,), (0,)), ((), ())),
        preferred_element_type=jnp.float32)
    cnt_g = jax.lax.dot_general(
        maskf, jnp.ones((N, 1), jnp.float32), (((0,), (0,)), ((), ())),
        preferred_element_type=jnp.float32)
    oc_ref[:, F:2 * F] = sum_g / jnp.maximum(cnt_g, 1.0)

    rows = [
        jnp.max(jnp.where(bcol == g, h2, neg), axis=0, keepdims=True)
        for g in range(G)
    ]
    oc_ref[:, 0:F] = jnp.concatenate(rows, axis=0)


def _sag_readout(h1, a0, a1, bcol, Wrel, brel, Wroot):
    return pl.pallas_call(
        _sag_body,
        grid=(1,),
        in_specs=[
            pl.BlockSpec((N, F), lambda i: (0, 0)),
            pl.BlockSpec((N, F), lambda i: (0, 0)),
            pl.BlockSpec((N, F), lambda i: (0, 0)),
            pl.BlockSpec((N, 1), lambda i: (0, 0)),
            pl.BlockSpec((F, 1), lambda i: (0, 0)),
            pl.BlockSpec((1, 1), lambda i: (0, 0)),
            pl.BlockSpec((F, 1), lambda i: (0, 0)),
        ],
        out_specs=[
            pl.BlockSpec((N, F), lambda i: (0, 0)),
            pl.BlockSpec((G, 2 * F), lambda i: (0, 0)),
        ],
        out_shape=[
            jax.ShapeDtypeStruct((N, F), jnp.float32),
            jax.ShapeDtypeStruct((G, 2 * F), jnp.float32),
        ],
    )(h1, a0, a1, bcol, Wrel, brel.reshape(1, 1), Wroot)


def _final_body(o_ref, w1_ref, b1_ref, w2_ref, b2_ref, w3_ref, b3_ref,
                out_ref):
    xs = o_ref[0] + o_ref[1] + o_ref[2] + o_ref[3]
    a = jnp.maximum(
        jnp.dot(xs, w1_ref[...], preferred_element_type=jnp.float32)
        + b1_ref[...], 0.0)
    a = jnp.maximum(
        jnp.dot(a, w2_ref[...], preferred_element_type=jnp.float32)
        + b2_ref[...], 0.0)
    z = (jnp.dot(a, w3_ref[...], preferred_element_type=jnp.float32)
         + b3_ref[...])
    zmax = jnp.max(z, axis=1, keepdims=True)
    zs = z - zmax
    lse = jnp.log(jnp.sum(jnp.exp(zs), axis=1, keepdims=True))
    out_ref[...] = zs - lse


def _final(outs, w1, b1, w2, b2, w3, b3):
    return pl.pallas_call(
        _final_body,
        grid=(1,),
        in_specs=[
            pl.BlockSpec((4, G, 2 * F), lambda i: (0, 0, 0)),
            pl.BlockSpec((2 * F, F), lambda i: (0, 0)),
            pl.BlockSpec((1, F), lambda i: (0, 0)),
            pl.BlockSpec((F, F), lambda i: (0, 0)),
            pl.BlockSpec((1, F), lambda i: (0, 0)),
            pl.BlockSpec((F, F), lambda i: (0, 0)),
            pl.BlockSpec((1, F), lambda i: (0, 0)),
        ],
        out_specs=pl.BlockSpec((G, F), lambda i: (0, 0)),
        out_shape=jax.ShapeDtypeStruct((G, F), jnp.float32),
    )(outs, w1, b1.reshape(1, F), w2, b2.reshape(1, F), w3,
      b3.reshape(1, F))


def kernel(x, edge_index, edge_attr, batch, emb_W, emb_b,
           c1_Wf, c1_bf, c1_Ws, c1_bs, p1_Wrel, p1_brel, p1_Wroot,
           c2_Wf, c2_bf, c2_Ws, c2_bs, p2_Wrel, p2_brel, p2_Wroot,
           c3_Wf, c3_bf, c3_Ws, c3_bs, p3_Wrel, p3_brel, p3_Wroot,
           c4_Wf, c4_bf, c4_Ws, c4_bs, p4_Wrel, p4_brel, p4_Wroot,
           lin1_W, lin1_b, lin2_W, lin2_b, lin3_W, lin3_b):
    src, dst = edge_index[0], edge_index[1]
    bcol = batch.reshape(N, 1)
    layers = [
        (c1_Wf, c1_bf, c1_Ws, c1_bs, p1_Wrel, p1_brel, p1_Wroot),
        (c2_Wf, c2_bf, c2_Ws, c2_bs, p2_Wrel, p2_brel, p2_Wroot),
        (c3_Wf, c3_bf, c3_Ws, c3_bs, p3_Wrel, p3_brel, p3_Wroot),
        (c4_Wf, c4_bf, c4_Ws, c4_bs, p4_Wrel, p4_brel, p4_Wroot),
    ]
    h = _emb(x, emb_W, emb_b)
    zeros_acc = jnp.zeros((N, F), jnp.float32)
    outs = []
    eh = E // 2
    d0, d1 = dst[:eh], dst[eh:]
    s0, s1 = src[:eh], src[eh:]
    ea0, ea1 = edge_attr[:eh], edge_attr[eh:]
    for (Wf, bf, Ws, bs, Wrel, brel, Wroot) in layers:
        hd0, hs0 = _sc_gather2(h, d0, s0, eh, 40)
        hd1, hs1 = _sc_gather2(h, d1, s1, eh, 40)
        m0 = _edge_messages(hd0, hs0, ea0, Wf, bf, Ws, bs, eh)
        m1 = _edge_messages(hd1, hs1, ea1, Wf, bf, Ws, bs, eh)
        pa = _sc_scatter_add(m0, d0, zeros_acc, eh, 40)
        pb = _sc_scatter_add(m1, d1, zeros_acc, eh, 40)
        h1 = _resid_relu(h, pa, pb)
        a = _sc_gather_scatter(h1, src, dst, zeros_acc)
        h, oc = _sag_readout(h1, a[0], a[1], bcol, Wrel, brel, Wroot)
        outs.append(oc)
    stacked = jnp.stack(outs, axis=0)
    return _final(stacked, lin1_W, lin1_b, lin2_W, lin2_b, lin3_W, lin3_b)


# speedup vs baseline: 1.0040x; 1.0040x over previous
import functools

import jax
import jax.numpy as jnp
from jax import lax
from jax.experimental import pallas as pl
from jax.experimental.pallas import tpu as pltpu
from jax.experimental.pallas import tpu_sc as plsc

N, E, F, D, G = 10000, 320000, 128, 16, 64

_NC, _NS = 2, 16
_NW = _NC * _NS
_CPW = E // _NW
_KG = 400
_NCH = _CPW // _KG
_ZPS = 624
_ZTAIL = N - _ZPS * _NS
_LANES = 16
_KS = 80
_NCHS = _CPW // _KS


def _sc_mesh():
    return plsc.VectorSubcoreMesh(
        core_axis_name="c", subcore_axis_name="s",
        num_cores=_NC, num_subcores=_NS)


def _sc_gather2(h, dst, src, ne, kg):
    cpw = ne // _NW
    nch = cpw // kg

    @functools.partial(
        pl.kernel,
        out_type=[jax.ShapeDtypeStruct((ne, F), jnp.float32),
                  jax.ShapeDtypeStruct((ne, F), jnp.float32)],
        mesh=_sc_mesh(),
        scratch_types=[
            pltpu.VMEM((kg,), jnp.int32),
            pltpu.VMEM((kg,), jnp.int32),
            pltpu.VMEM((kg, F), jnp.float32),
            pltpu.VMEM((kg, F), jnp.float32),
            pltpu.SemaphoreType.DMA,
            pltpu.SemaphoreType.DMA,
            pltpu.SemaphoreType.DMA,
            pltpu.SemaphoreType.DMA,
        ],
    )
    def k(h_hbm, d_hbm, s_hbm, od_hbm, os_hbm, idxd, idxs, bufd, bufs,
          semgd, semgs, semod, semos):
        wid = lax.axis_index("s") * _NC + lax.axis_index("c")
        base = wid * cpw

        pltpu.sync_copy(d_hbm.at[pl.ds(base, kg)], idxd)
        pltpu.async_copy(h_hbm.at[idxd], bufd, semgd)
        pltpu.sync_copy(s_hbm.at[pl.ds(base, kg)], idxs)
        pltpu.async_copy(h_hbm.at[idxs], bufs, semgs)

        @pl.loop(0, nch)
        def _(j):
            off = base + j * kg
            pltpu.make_async_copy(h_hbm.at[idxd], bufd, semgd).wait()
            pltpu.async_copy(bufd, od_hbm.at[pl.ds(off, kg)], semod)
            pltpu.make_async_copy(h_hbm.at[idxs], bufs, semgs).wait()
            pltpu.async_copy(bufs, os_hbm.at[pl.ds(off, kg)], semos)

            @pl.when(j < nch - 1)
            def _():
                nxt = off + kg
                pltpu.sync_copy(d_hbm.at[pl.ds(nxt, kg)], idxd)
                pltpu.make_async_copy(
                    bufd, od_hbm.at[pl.ds(off, kg)], semod).wait()
                pltpu.async_copy(h_hbm.at[idxd], bufd, semgd)
                pltpu.sync_copy(s_hbm.at[pl.ds(nxt, kg)], idxs)
                pltpu.make_async_copy(
                    bufs, os_hbm.at[pl.ds(off, kg)], semos).wait()
                pltpu.async_copy(h_hbm.at[idxs], bufs, semgs)

        last = base + (nch - 1) * kg
        pltpu.make_async_copy(bufd, od_hbm.at[pl.ds(last, kg)], semod).wait()
        pltpu.make_async_copy(bufs, os_hbm.at[pl.ds(last, kg)], semos).wait()

    return k(h, dst, src)


def _zero_acc(z_hbm, acc, sid):
    row0 = sid * _ZPS
    pltpu.sync_copy(z_hbm.at[pl.ds(row0, _ZPS)], acc.at[pl.ds(row0, _ZPS)])

    @pl.when(sid == _NS - 1)
    def _():
        pltpu.sync_copy(z_hbm.at[pl.ds(_ZPS * _NS, _ZTAIL)],
                        acc.at[pl.ds(_ZPS * _NS, _ZTAIL)])


def _writeback(acc, out_hbm, cid, sid):
    row0 = sid * _ZPS
    pltpu.sync_copy(acc.at[pl.ds(row0, _ZPS)],
                    out_hbm.at[cid].at[pl.ds(row0, _ZPS)])

    @pl.when(sid == _NS - 1)
    def _():
        pltpu.sync_copy(
            acc.at[pl.ds(_ZPS * _NS, _ZTAIL)],
            out_hbm.at[cid].at[pl.ds(_ZPS * _NS, _ZTAIL)])


def _sc_scatter_add(m, dst, zeros_acc, ne, ks):
    cpw = ne // _NW
    nchs = cpw // ks

    @functools.partial(
        pl.kernel,
        out_type=jax.ShapeDtypeStruct((_NC, N, F), jnp.float32),
        mesh=_sc_mesh(),
        scratch_types=[
            [pltpu.VMEM((ks,), jnp.int32)] * 2,
            [pltpu.VMEM((ks, F), jnp.float32)] * 2,
            pltpu.VMEM_SHARED((N, F), jnp.float32),
            [pltpu.SemaphoreType.DMA] * 2,
        ],
    )
    def k(m_hbm, d_hbm, z_hbm, out_hbm, idx, buf, acc, semm):
        cid = lax.axis_index("c")
        sid = lax.axis_index("s")
        base = (sid * _NC + cid) * cpw

        _zero_acc(z_hbm, acc, sid)
        plsc.subcore_barrier()

        for b in (0, 1):
            pltpu.async_copy(m_hbm.at[pl.ds(base + b * ks, ks)],
                             buf[b], semm[b])

        def step(j, b):
            off = base + j * ks
            pltpu.sync_copy(d_hbm.at[pl.ds(off, ks)], idx[b])
            pltpu.make_async_copy(
                m_hbm.at[pl.ds(off, ks)], buf[b], semm[b]).wait()
            pltpu.sync_copy(buf[b], acc.at[idx[b]], add=True)

            @pl.when(j + 2 < nchs)
            def _():
                pltpu.async_copy(
                    m_hbm.at[pl.ds(off + 2 * ks, ks)], buf[b], semm[b])

        @pl.loop(0, nchs // 2)
        def _(t):
            step(2 * t, 0)
            step(2 * t + 1, 1)

        if nchs % 2:
            step(nchs - 1, 0)

        plsc.subcore_barrier()
        _writeback(acc, out_hbm, cid, sid)

    return k(m, dst, zeros_acc)


def _sc_gather_scatter(h1, src, dst, zeros_acc):
    @functools.partial(
        pl.kernel,
        out_type=jax.ShapeDtypeStruct((_NC, N, F), jnp.float32),
        mesh=_sc_mesh(),
        scratch_types=[
            [pltpu.VMEM((_KS,), jnp.int32)] * 2,
            pltpu.VMEM((_KS,), jnp.int32),
            [pltpu.VMEM((_KS, F), jnp.float32)] * 2,
            pltpu.VMEM_SHARED((N, F), jnp.float32),
            [pltpu.SemaphoreType.DMA] * 2,
        ],
    )
    def k(h_hbm, s_hbm, d_hbm, z_hbm, out_hbm, idxs, idxd, buf, acc, semg):
        cid = lax.axis_index("c")
        sid = lax.axis_index("s")
        base = (sid * _NC + cid) * _CPW

        _zero_acc(z_hbm, acc, sid)
        plsc.subcore_barrier()

        for b in (0, 1):
            pltpu.sync_copy(s_hbm.at[pl.ds(base + b * _KS, _KS)], idxs[b])
            pltpu.async_copy(h_hbm.at[idxs[b]], buf[b], semg[b])

        def step(j, b):
            off = base + j * _KS
            pltpu.sync_copy(d_hbm.at[pl.ds(off, _KS)], idxd)
            pltpu.make_async_copy(
                h_hbm.at[idxs[b]], buf[b], semg[b]).wait()
            pltpu.sync_copy(buf[b], acc.at[idxd], add=True)

            @pl.when(j + 2 < _NCHS)
            def _():
                pltpu.sync_copy(
                    s_hbm.at[pl.ds(off + 2 * _KS, _KS)], idxs[b])
                pltpu.async_copy(h_hbm.at[idxs[b]], buf[b], semg[b])

        @pl.loop(0, _NCHS // 2)
        def _(t):
            step(2 * t, 0)
            step(2 * t + 1, 1)

        step(_NCHS - 1, 0)

        plsc.subcore_barrier()
        _writeback(acc, out_hbm, cid, sid)

    return k(h1, src, dst, zeros_acc)


_BN = 2000
_BE = 2000


def _sigmoid(x):
    return 1.0 / (1.0 + jnp.exp(-x))


def _softplus(x):
    return jnp.maximum(x, 0.0) + jnp.log(1.0 + jnp.exp(-jnp.abs(x)))


def _emb_body(x_ref, w_ref, b_ref, o_ref):
    o_ref[...] = (
        jnp.dot(x_ref[...], w_ref[...], preferred_element_type=jnp.float32)
        + b_ref[...]
    )


def _emb(x, W, b):
    return pl.pallas_call(
        _emb_body,
        grid=(N // _BN,),
        in_specs=[
            pl.BlockSpec((_BN, F), lambda i: (i, 0)),
            pl.BlockSpec((F, F), lambda i: (0, 0)),
            pl.BlockSpec((1, F), lambda i: (0, 0)),
        ],
        out_specs=pl.BlockSpec((_BN, F), lambda i: (i, 0)),
        out_shape=jax.ShapeDtypeStruct((N, F), jnp.float32),
    )(x, W, b.reshape(1, F))


def _edge_body(hd_ref, hs_ref, e_ref, wf_ref, bf_ref, ws_ref, bs_ref, m_ref):
    hd = hd_ref[...]
    hs = hs_ref[...]
    ea = e_ref[...]

    def proj(w_ref, b_ref):
        return (
            jnp.dot(hd, w_ref[0:F, :], preferred_element_type=jnp.float32)
            + jnp.dot(hs, w_ref[F:2 * F, :], preferred_element_type=jnp.float32)
            + jnp.dot(ea, w_ref[2 * F:2 * F + D, :],
                      preferred_element_type=jnp.float32)
            + b_ref[...]
        )

    f = proj(wf_ref, bf_ref)
    s = proj(ws_ref, bs_ref)
    m_ref[...] = _sigmoid(f) * _softplus(s)


def _edge_messages(hd, hs, ea, Wf, bf, Ws, bs, ne):
    return pl.pallas_call(
        _edge_body,
        grid=(ne // _BE,),
        in_specs=[
            pl.BlockSpec((_BE, F), lambda i: (i, 0)),
            pl.BlockSpec((_BE, F), lambda i: (i, 0)),
            pl.BlockSpec((_BE, D), lambda i: (i, 0)),
            pl.BlockSpec((2 * F + D, F), lambda i: (0, 0)),
            pl.BlockSpec((1, F), lambda i: (0, 0)),
            pl.BlockSpec((2 * F + D, F), lambda i: (0, 0)),
            pl.BlockSpec((1, F), lambda i: (0, 0)),
        ],
        out_specs=pl.BlockSpec((_BE, F), lambda i: (i, 0)),
        out_shape=jax.ShapeDtypeStruct((ne, F), jnp.float32),
    )(hd, hs, ea, Wf, bf.reshape(1, F), Ws, bs.reshape(1, F))


def _resid_body(h_ref, p0_ref, p1_ref, p2_ref, p3_ref, o_ref):
    o_ref[...] = jnp.maximum(
        h_ref[...] + (p0_ref[...] + p1_ref[...])
        + (p2_ref[...] + p3_ref[...]), 0.0)


def _resid_relu(h, pa, pb):
    return pl.pallas_call(
        _resid_body,
        grid=(N // _BN,),
        in_specs=[pl.BlockSpec((_BN, F), lambda i: (i, 0))] * 5,
        out_specs=pl.BlockSpec((_BN, F), lambda i: (i, 0)),
        out_shape=jax.ShapeDtypeStruct((N, F), jnp.float32),
    )(h, pa[0], pa[1], pb[0], pb[1])


def _sag_body(h1_ref, a0_ref, a1_ref, b_ref, wrel_ref, brel_ref, wroot_ref,
              h2_ref, oc_ref):
    h1 = h1_ref[...]
    agg = a0_ref[...] + a1_ref[...]
    bcol = b_ref[...]
    s = (
        jnp.dot(agg, wrel_ref[...], preferred_element_type=jnp.float32)
        + brel_ref[...]
        + jnp.dot(h1, wroot_ref[...], preferred_element_type=jnp.float32)
    )
    seg = jax.lax.broadcasted_iota(jnp.int32, (N, G), 1)
    mask = bcol == seg
    maskf = mask.astype(jnp.float32)
    neg = jnp.float32(-1e30)
    smax = jnp.max(jnp.where(mask, s, neg), axis=0, keepdims=True)
    smax_n = jnp.sum(maskf * smax, axis=1, keepdims=True)
    ex = jnp.exp(s - smax_n)
    den = jnp.sum(maskf * ex, axis=0, keepdims=True)
    den_n = jnp.sum(maskf * den, axis=1, keepdims=True)
    h2 = h1 * (ex / den_n)
    h2_ref[...] = h2
    sum_g = jax.lax.dot_general(
        maskf, h2, (((0,), (0,)), ((), ())),
        preferred_element_type=jnp.float32)
    cnt_g = jax.lax.dot_general(
        maskf, jnp.ones((N, 1), jnp.float32), (((0,), (0,)), ((), ())),
        preferred_element_type=jnp.float32)
    oc_ref[:, F:2 * F] = sum_g / jnp.maximum(cnt_g, 1.0)

    rows = [
        jnp.max(jnp.where(bcol == g, h2, neg), axis=0, keepdims=True)
        for g in range(G)
    ]
    oc_ref[:, 0:F] = jnp.concatenate(rows, axis=0)


def _sag_readout(h1, a0, a1, bcol, Wrel, brel, Wroot):
    return pl.pallas_call(
        _sag_body,
        grid=(1,),
        in_specs=[
            pl.BlockSpec((N, F), lambda i: (0, 0)),
            pl.BlockSpec((N, F), lambda i: (0, 0)),
            pl.BlockSpec((N, F), lambda i: (0, 0)),
            pl.BlockSpec((N, 1), lambda i: (0, 0)),
            pl.BlockSpec((F, 1), lambda i: (0, 0)),
            pl.BlockSpec((1, 1), lambda i: (0, 0)),
            pl.BlockSpec((F, 1), lambda i: (0, 0)),
        ],
        out_specs=[
            pl.BlockSpec((N, F), lambda i: (0, 0)),
            pl.BlockSpec((G, 2 * F), lambda i: (0, 0)),
        ],
        out_shape=[
            jax.ShapeDtypeStruct((N, F), jnp.float32),
            jax.ShapeDtypeStruct((G, 2 * F), jnp.float32),
        ],
    )(h1, a0, a1, bcol, Wrel, brel.reshape(1, 1), Wroot)


def _final_body(o_ref, w1_ref, b1_ref, w2_ref, b2_ref, w3_ref, b3_ref,
                out_ref):
    xs = o_ref[0] + o_ref[1] + o_ref[2] + o_ref[3]
    a = jnp.maximum(
        jnp.dot(xs, w1_ref[...], preferred_element_type=jnp.float32)
        + b1_ref[...], 0.0)
    a = jnp.maximum(
        jnp.dot(a, w2_ref[...], preferred_element_type=jnp.float32)
        + b2_ref[...], 0.0)
    z = (jnp.dot(a, w3_ref[...], preferred_element_type=jnp.float32)
         + b3_ref[...])
    zmax = jnp.max(z, axis=1, keepdims=True)
    zs = z - zmax
    lse = jnp.log(jnp.sum(jnp.exp(zs), axis=1, keepdims=True))
    out_ref[...] = zs - lse


def _final(outs, w1, b1, w2, b2, w3, b3):
    return pl.pallas_call(
        _final_body,
        grid=(1,),
        in_specs=[
            pl.BlockSpec((4, G, 2 * F), lambda i: (0, 0, 0)),
            pl.BlockSpec((2 * F, F), lambda i: (0, 0)),
            pl.BlockSpec((1, F), lambda i: (0, 0)),
            pl.BlockSpec((F, F), lambda i: (0, 0)),
            pl.BlockSpec((1, F), lambda i: (0, 0)),
            pl.BlockSpec((F, F), lambda i: (0, 0)),
            pl.BlockSpec((1, F), lambda i: (0, 0)),
        ],
        out_specs=pl.BlockSpec((G, F), lambda i: (0, 0)),
        out_shape=jax.ShapeDtypeStruct((G, F), jnp.float32),
    )(outs, w1, b1.reshape(1, F), w2, b2.reshape(1, F), w3,
      b3.reshape(1, F))


def kernel(x, edge_index, edge_attr, batch, emb_W, emb_b,
           c1_Wf, c1_bf, c1_Ws, c1_bs, p1_Wrel, p1_brel, p1_Wroot,
           c2_Wf, c2_bf, c2_Ws, c2_bs, p2_Wrel, p2_brel, p2_Wroot,
           c3_Wf, c3_bf, c3_Ws, c3_bs, p3_Wrel, p3_brel, p3_Wroot,
           c4_Wf, c4_bf, c4_Ws, c4_bs, p4_Wrel, p4_brel, p4_Wroot,
           lin1_W, lin1_b, lin2_W, lin2_b, lin3_W, lin3_b):
    src, dst = edge_index[0], edge_index[1]
    bcol = batch.reshape(N, 1)
    layers = [
        (c1_Wf, c1_bf, c1_Ws, c1_bs, p1_Wrel, p1_brel, p1_Wroot),
        (c2_Wf, c2_bf, c2_Ws, c2_bs, p2_Wrel, p2_brel, p2_Wroot),
        (c3_Wf, c3_bf, c3_Ws, c3_bs, p3_Wrel, p3_brel, p3_Wroot),
        (c4_Wf, c4_bf, c4_Ws, c4_bs, p4_Wrel, p4_brel, p4_Wroot),
    ]
    h = _emb(x, emb_W, emb_b)
    zeros_acc = jnp.zeros((N, F), jnp.float32)
    outs = []
    eh = E // 2
    d0, d1 = dst[:eh], dst[eh:]
    s0, s1 = src[:eh], src[eh:]
    ea0, ea1 = edge_attr[:eh], edge_attr[eh:]
    for (Wf, bf, Ws, bs, Wrel, brel, Wroot) in layers:
        hd0, hs0 = _sc_gather2(h, d0, s0, eh, 200)
        hd1, hs1 = _sc_gather2(h, d1, s1, eh, 200)
        m0 = _edge_messages(hd0, hs0, ea0, Wf, bf, Ws, bs, eh)
        m1 = _edge_messages(hd1, hs1, ea1, Wf, bf, Ws, bs, eh)
        pa = _sc_scatter_add(m0, d0, zeros_acc, eh, 40)
        pb = _sc_scatter_add(m1, d1, zeros_acc, eh, 40)
        h1 = _resid_relu(h, pa, pb)
        a = _sc_gather_scatter(h1, src, dst, zeros_acc)
        h, oc = _sag_readout(h1, a[0], a[1], bcol, Wrel, brel, Wroot)
        outs.append(oc)
    stacked = jnp.stack(outs, axis=0)
    return _final(stacked, lin1_W, lin1_b, lin2_W, lin2_b, lin3_W, lin3_b)


# speedup vs baseline: 1.0043x; 1.0003x over previous
import functools

import jax
import jax.numpy as jnp
from jax import lax
from jax.experimental import pallas as pl
from jax.experimental.pallas import tpu as pltpu
from jax.experimental.pallas import tpu_sc as plsc

N, E, F, D, G = 10000, 320000, 128, 16, 64

_NC, _NS = 2, 16
_NW = _NC * _NS
_CPW = E // _NW
_ZPS = 624
_ZTAIL = N - _ZPS * _NS
_LANES = 16
_KS = 80
_NCHS = _CPW // _KS


def _sc_mesh():
    return plsc.VectorSubcoreMesh(
        core_axis_name="c", subcore_axis_name="s",
        num_cores=_NC, num_subcores=_NS)


def _sc_gather2(h, dst, src, ne, kg):
    cpw = ne // _NW
    nch = cpw // kg

    @functools.partial(
        pl.kernel,
        out_type=[jax.ShapeDtypeStruct((ne, F), jnp.float32),
                  jax.ShapeDtypeStruct((ne, F), jnp.float32)],
        mesh=_sc_mesh(),
        scratch_types=[
            pltpu.VMEM((kg,), jnp.int32),
            pltpu.VMEM((kg,), jnp.int32),
            pltpu.VMEM((kg, F), jnp.float32),
            pltpu.VMEM((kg, F), jnp.float32),
            pltpu.SemaphoreType.DMA,
            pltpu.SemaphoreType.DMA,
            pltpu.SemaphoreType.DMA,
            pltpu.SemaphoreType.DMA,
        ],
    )
    def k(h_hbm, d_hbm, s_hbm, od_hbm, os_hbm, idxd, idxs, bufd, bufs,
          semgd, semgs, semod, semos):
        wid = lax.axis_index("s") * _NC + lax.axis_index("c")
        base = wid * cpw

        pltpu.sync_copy(d_hbm.at[pl.ds(base, kg)], idxd)
        pltpu.async_copy(h_hbm.at[idxd], bufd, semgd)
        pltpu.sync_copy(s_hbm.at[pl.ds(base, kg)], idxs)
        pltpu.async_copy(h_hbm.at[idxs], bufs, semgs)

        @pl.loop(0, nch)
        def _(j):
            off = base + j * kg
            pltpu.make_async_copy(h_hbm.at[idxd], bufd, semgd).wait()
            pltpu.async_copy(bufd, od_hbm.at[pl.ds(off, kg)], semod)
            pltpu.make_async_copy(h_hbm.at[idxs], bufs, semgs).wait()
            pltpu.async_copy(bufs, os_hbm.at[pl.ds(off, kg)], semos)

            @pl.when(j < nch - 1)
            def _():
                nxt = off + kg
                pltpu.sync_copy(d_hbm.at[pl.ds(nxt, kg)], idxd)
                pltpu.make_async_copy(
                    bufd, od_hbm.at[pl.ds(off, kg)], semod).wait()
                pltpu.async_copy(h_hbm.at[idxd], bufd, semgd)
                pltpu.sync_copy(s_hbm.at[pl.ds(nxt, kg)], idxs)
                pltpu.make_async_copy(
                    bufs, os_hbm.at[pl.ds(off, kg)], semos).wait()
                pltpu.async_copy(h_hbm.at[idxs], bufs, semgs)

        last = base + (nch - 1) * kg
        pltpu.make_async_copy(bufd, od_hbm.at[pl.ds(last, kg)], semod).wait()
        pltpu.make_async_copy(bufs, os_hbm.at[pl.ds(last, kg)], semos).wait()

    return k(h, dst, src)


def _zero_acc(z_hbm, acc, sid):
    row0 = sid * _ZPS
    pltpu.sync_copy(z_hbm.at[pl.ds(row0, _ZPS)], acc.at[pl.ds(row0, _ZPS)])

    @pl.when(sid == _NS - 1)
    def _():
        pltpu.sync_copy(z_hbm.at[pl.ds(_ZPS * _NS, _ZTAIL)],
                        acc.at[pl.ds(_ZPS * _NS, _ZTAIL)])


def _writeback(acc, out_hbm, cid, sid):
    row0 = sid * _ZPS
    pltpu.sync_copy(acc.at[pl.ds(row0, _ZPS)],
                    out_hbm.at[cid].at[pl.ds(row0, _ZPS)])

    @pl.when(sid == _NS - 1)
    def _():
        pltpu.sync_copy(
            acc.at[pl.ds(_ZPS * _NS, _ZTAIL)],
            out_hbm.at[cid].at[pl.ds(_ZPS * _NS, _ZTAIL)])


def _sc_scatter_add(m, dst, zeros_acc, ne, ks):
    cpw = ne // _NW
    nchs = cpw // ks

    @functools.partial(
        pl.kernel,
        out_type=jax.ShapeDtypeStruct((_NC, N, F), jnp.float32),
        mesh=_sc_mesh(),
        scratch_types=[
            [pltpu.VMEM((ks,), jnp.int32)] * 2,
            [pltpu.VMEM((ks, F), jnp.float32)] * 2,
            pltpu.VMEM_SHARED((N, F), jnp.float32),
            [pltpu.SemaphoreType.DMA] * 2,
        ],
    )
    def k(m_hbm, d_hbm, z_hbm, out_hbm, idx, buf, acc, semm):
        cid = lax.axis_index("c")
        sid = lax.axis_index("s")
        base = (sid * _NC + cid) * cpw

        _zero_acc(z_hbm, acc, sid)
        plsc.subcore_barrier()

        for b in (0, 1):
            pltpu.async_copy(m_hbm.at[pl.ds(base + b * ks, ks)],
                             buf[b], semm[b])

        def step(j, b):
            off = base + j * ks
            pltpu.sync_copy(d_hbm.at[pl.ds(off, ks)], idx[b])
            pltpu.make_async_copy(
                m_hbm.at[pl.ds(off, ks)], buf[b], semm[b]).wait()
            pltpu.sync_copy(buf[b], acc.at[idx[b]], add=True)

            @pl.when(j + 2 < nchs)
            def _():
                pltpu.async_copy(
                    m_hbm.at[pl.ds(off + 2 * ks, ks)], buf[b], semm[b])

        @pl.loop(0, nchs // 2)
        def _(t):
            step(2 * t, 0)
            step(2 * t + 1, 1)

        if nchs % 2:
            step(nchs - 1, 0)

        plsc.subcore_barrier()
        _writeback(acc, out_hbm, cid, sid)

    return k(m, dst, zeros_acc)


def _sc_gather_scatter(h1, src, dst, zeros_acc):
    @functools.partial(
        pl.kernel,
        out_type=jax.ShapeDtypeStruct((_NC, N, F), jnp.float32),
        mesh=_sc_mesh(),
        scratch_types=[
            [pltpu.VMEM((_KS,), jnp.int32)] * 2,
            pltpu.VMEM((_KS,), jnp.int32),
            [pltpu.VMEM((_KS, F), jnp.float32)] * 2,
            pltpu.VMEM_SHARED((N, F), jnp.float32),
            [pltpu.SemaphoreType.DMA] * 2,
        ],
    )
    def k(h_hbm, s_hbm, d_hbm, z_hbm, out_hbm, idxs, idxd, buf, acc, semg):
        cid = lax.axis_index("c")
        sid = lax.axis_index("s")
        base = (sid * _NC + cid) * _CPW

        _zero_acc(z_hbm, acc, sid)
        plsc.subcore_barrier()

        for b in (0, 1):
            pltpu.sync_copy(s_hbm.at[pl.ds(base + b * _KS, _KS)], idxs[b])
            pltpu.async_copy(h_hbm.at[idxs[b]], buf[b], semg[b])

        def step(j, b):
            off = base + j * _KS
            pltpu.sync_copy(d_hbm.at[pl.ds(off, _KS)], idxd)
            pltpu.make_async_copy(
                h_hbm.at[idxs[b]], buf[b], semg[b]).wait()
            pltpu.sync_copy(buf[b], acc.at[idxd], add=True)

            @pl.when(j + 2 < _NCHS)
            def _():
                pltpu.sync_copy(
                    s_hbm.at[pl.ds(off + 2 * _KS, _KS)], idxs[b])
                pltpu.async_copy(h_hbm.at[idxs[b]], buf[b], semg[b])

        @pl.loop(0, _NCHS // 2)
        def _(t):
            step(2 * t, 0)
            step(2 * t + 1, 1)

        step(_NCHS - 1, 0)

        plsc.subcore_barrier()
        _writeback(acc, out_hbm, cid, sid)

    return k(h1, src, dst, zeros_acc)


_BN = 2000
_BE = 2000


def _sigmoid(x):
    return 1.0 / (1.0 + jnp.exp(-x))


def _softplus(x):
    return jnp.maximum(x, 0.0) + jnp.log(1.0 + jnp.exp(-jnp.abs(x)))


def _emb_body(x_ref, w_ref, b_ref, o_ref):
    o_ref[...] = (
        jnp.dot(x_ref[...], w_ref[...], preferred_element_type=jnp.float32)
        + b_ref[...]
    )


def _emb(x, W, b):
    return pl.pallas_call(
        _emb_body,
        grid=(N // _BN,),
        in_specs=[
            pl.BlockSpec((_BN, F), lambda i: (i, 0)),
            pl.BlockSpec((F, F), lambda i: (0, 0)),
            pl.BlockSpec((1, F), lambda i: (0, 0)),
        ],
        out_specs=pl.BlockSpec((_BN, F), lambda i: (i, 0)),
        out_shape=jax.ShapeDtypeStruct((N, F), jnp.float32),
    )(x, W, b.reshape(1, F))


def _edge_body(hd_ref, hs_ref, e_ref, wf_ref, bf_ref, ws_ref, bs_ref, m_ref):
    hd = hd_ref[...]
    hs = hs_ref[...]
    ea = e_ref[...]

    def proj(w_ref, b_ref):
        return (
            jnp.dot(hd, w_ref[0:F, :], preferred_element_type=jnp.float32)
            + jnp.dot(hs, w_ref[F:2 * F, :], preferred_element_type=jnp.float32)
            + jnp.dot(ea, w_ref[2 * F:2 * F + D, :],
                      preferred_element_type=jnp.float32)
            + b_ref[...]
        )

    f = proj(wf_ref, bf_ref)
    s = proj(ws_ref, bs_ref)
    m_ref[...] = _sigmoid(f) * _softplus(s)


def _edge_messages(hd, hs, ea, Wf, bf, Ws, bs, ne):
    return pl.pallas_call(
        _edge_body,
        grid=(ne // _BE,),
        in_specs=[
            pl.BlockSpec((_BE, F), lambda i: (i, 0)),
            pl.BlockSpec((_BE, F), lambda i: (i, 0)),
            pl.BlockSpec((_BE, D), lambda i: (i, 0)),
            pl.BlockSpec((2 * F + D, F), lambda i: (0, 0)),
            pl.BlockSpec((1, F), lambda i: (0, 0)),
            pl.BlockSpec((2 * F + D, F), lambda i: (0, 0)),
            pl.BlockSpec((1, F), lambda i: (0, 0)),
        ],
        out_specs=pl.BlockSpec((_BE, F), lambda i: (i, 0)),
        out_shape=jax.ShapeDtypeStruct((ne, F), jnp.float32),
    )(hd, hs, ea, Wf, bf.reshape(1, F), Ws, bs.reshape(1, F))


def _resid_body(h_ref, p0_ref, p1_ref, p2_ref, p3_ref, o_ref):
    o_ref[...] = jnp.maximum(
        h_ref[...] + (p0_ref[...] + p1_ref[...])
        + (p2_ref[...] + p3_ref[...]), 0.0)


def _resid_relu(h, pa, pb):
    return pl.pallas_call(
        _resid_body,
        grid=(N // _BN,),
        in_specs=[pl.BlockSpec((_BN, F), lambda i: (i, 0))] * 5,
        out_specs=pl.BlockSpec((_BN, F), lambda i: (i, 0)),
        out_shape=jax.ShapeDtypeStruct((N, F), jnp.float32),
    )(h, pa[0], pa[1], pb[0], pb[1])


def _sag_body(h1_ref, a0_ref, a1_ref, b_ref, wrel_ref, brel_ref, wroot_ref,
              h2_ref, oc_ref):
    h1 = h1_ref[...]
    agg = a0_ref[...] + a1_ref[...]
    bcol = b_ref[...]
    s = (
        jnp.dot(agg, wrel_ref[...], preferred_element_type=jnp.float32)
        + brel_ref[...]
        + jnp.dot(h1, wroot_ref[...], preferred_element_type=jnp.float32)
    )
    seg = jax.lax.broadcasted_iota(jnp.int32, (N, G), 1)
    mask = bcol == seg
    maskf = mask.astype(jnp.float32)
    neg = jnp.float32(-1e30)
    smax = jnp.max(jnp.where(mask, s, neg), axis=0, keepdims=True)
    smax_n = jnp.sum(maskf * smax, axis=1, keepdims=True)
    ex = jnp.exp(s - smax_n)
    den = jnp.sum(maskf * ex, axis=0, keepdims=True)
    den_n = jnp.sum(maskf * den, axis=1, keepdims=True)
    h2 = h1 * (ex / den_n)
    h2_ref[...] = h2
    sum_g = jax.lax.dot_general(
        maskf, h2, (((0,), (0,)), ((), ())),
        preferred_element_type=jnp.float32)
    cnt_g = jax.lax.dot_general(
        maskf, jnp.ones((N, 1), jnp.float32), (((0,), (0,)), ((), ())),
        preferred_element_type=jnp.float32)
    oc_ref[:, F:2 * F] = sum_g / jnp.maximum(cnt_g, 1.0)

    rows = [
        jnp.max(jnp.where(bcol == g, h2, neg), axis=0, keepdims=True)
        for g in range(G)
    ]
    oc_ref[:, 0:F] = jnp.concatenate(rows, axis=0)


def _sag_readout(h1, a0, a1, bcol, Wrel, brel, Wroot):
    return pl.pallas_call(
        _sag_body,
        grid=(1,),
        in_specs=[
            pl.BlockSpec((N, F), lambda i: (0, 0)),
            pl.BlockSpec((N, F), lambda i: (0, 0)),
            pl.BlockSpec((N, F), lambda i: (0, 0)),
            pl.BlockSpec((N, 1), lambda i: (0, 0)),
            pl.BlockSpec((F, 1), lambda i: (0, 0)),
            pl.BlockSpec((1, 1), lambda i: (0, 0)),
            pl.BlockSpec((F, 1), lambda i: (0, 0)),
        ],
        out_specs=[
            pl.BlockSpec((N, F), lambda i: (0, 0)),
            pl.BlockSpec((G, 2 * F), lambda i: (0, 0)),
        ],
        out_shape=[
            jax.ShapeDtypeStruct((N, F), jnp.float32),
            jax.ShapeDtypeStruct((G, 2 * F), jnp.float32),
        ],
    )(h1, a0, a1, bcol, Wrel, brel.reshape(1, 1), Wroot)


def _final_body(o_ref, w1_ref, b1_ref, w2_ref, b2_ref, w3_ref, b3_ref,
                out_ref):
    xs = o_ref[0] + o_ref[1] + o_ref[2] + o_ref[3]
    a = jnp.maximum(
        jnp.dot(xs, w1_ref[...], preferred_element_type=jnp.float32)
        + b1_ref[...], 0.0)
    a = jnp.maximum(
        jnp.dot(a, w2_ref[...], preferred_element_type=jnp.float32)
        + b2_ref[...], 0.0)
    z = (jnp.dot(a, w3_ref[...], preferred_element_type=jnp.float32)
         + b3_ref[...])
    zmax = jnp.max(z, axis=1, keepdims=True)
    zs = z - zmax
    lse = jnp.log(jnp.sum(jnp.exp(zs), axis=1, keepdims=True))
    out_ref[...] = zs - lse


def _final(outs, w1, b1, w2, b2, w3, b3):
    return pl.pallas_call(
        _final_body,
        grid=(1,),
        in_specs=[
            pl.BlockSpec((4, G, 2 * F), lambda i: (0, 0, 0)),
            pl.BlockSpec((2 * F, F), lambda i: (0, 0)),
            pl.BlockSpec((1, F), lambda i: (0, 0)),
            pl.BlockSpec((F, F), lambda i: (0, 0)),
            pl.BlockSpec((1, F), lambda i: (0, 0)),
            pl.BlockSpec((F, F), lambda i: (0, 0)),
            pl.BlockSpec((1, F), lambda i: (0, 0)),
        ],
        out_specs=pl.BlockSpec((G, F), lambda i: (0, 0)),
        out_shape=jax.ShapeDtypeStruct((G, F), jnp.float32),
    )(outs, w1, b1.reshape(1, F), w2, b2.reshape(1, F), w3,
      b3.reshape(1, F))


def kernel(x, edge_index, edge_attr, batch, emb_W, emb_b,
           c1_Wf, c1_bf, c1_Ws, c1_bs, p1_Wrel, p1_brel, p1_Wroot,
           c2_Wf, c2_bf, c2_Ws, c2_bs, p2_Wrel, p2_brel, p2_Wroot,
           c3_Wf, c3_bf, c3_Ws, c3_bs, p3_Wrel, p3_brel, p3_Wroot,
           c4_Wf, c4_bf, c4_Ws, c4_bs, p4_Wrel, p4_brel, p4_Wroot,
           lin1_W, lin1_b, lin2_W, lin2_b, lin3_W, lin3_b):
    src, dst = edge_index[0], edge_index[1]
    bcol = batch.reshape(N, 1)
    layers = [
        (c1_Wf, c1_bf, c1_Ws, c1_bs, p1_Wrel, p1_brel, p1_Wroot),
        (c2_Wf, c2_bf, c2_Ws, c2_bs, p2_Wrel, p2_brel, p2_Wroot),
        (c3_Wf, c3_bf, c3_Ws, c3_bs, p3_Wrel, p3_brel, p3_Wroot),
        (c4_Wf, c4_bf, c4_Ws, c4_bs, p4_Wrel, p4_brel, p4_Wroot),
    ]
    h = _emb(x, emb_W, emb_b)
    zeros_acc = jnp.zeros((N, F), jnp.float32)
    outs = []
    eh = E // 2
    d0, d1 = dst[:eh], dst[eh:]
    s0, s1 = src[:eh], src[eh:]
    ea0, ea1 = edge_attr[:eh], edge_attr[eh:]
    for (Wf, bf, Ws, bs, Wrel, brel, Wroot) in layers:
        hd0, hs0 = _sc_gather2(h, d0, s0, eh, 200)
        hd1, hs1 = _sc_gather2(h, d1, s1, eh, 200)
        m0 = _edge_messages(hd0, hs0, ea0, Wf, bf, Ws, bs, eh)
        m1 = _edge_messages(hd1, hs1, ea1, Wf, bf, Ws, bs, eh)
        pa = _sc_scatter_add(m0, d0, zeros_acc, eh, 40)
        pb = _sc_scatter_add(m1, d1, zeros_acc, eh, 40)
        h1 = _resid_relu(h, pa, pb)
        a = _sc_gather_scatter(h1, src, dst, zeros_acc)
        h, oc = _sag_readout(h1, a[0], a[1], bcol, Wrel, brel, Wroot)
        outs.append(oc)
    stacked = jnp.stack(outs, axis=0)
    return _final(stacked, lin1_W, lin1_b, lin2_W, lin2_b, lin3_W, lin3_b)
